# Initial kernel scaffold; baseline (speedup 1.0000x reference)
import jax
import jax.numpy as jnp
from jax.experimental import pallas as pl


def kernel(node_feature, edge_index, node2graph, h_inds, t_inds, labels, conv0_Wa, conv0_ba, conv0_Wb, conv0_bb, conv1_Wa, conv1_ba, conv1_Wb, conv1_bb, conv2_Wa, conv2_ba, conv2_Wb, conv2_bb, fc1_W, fc1_b, ln1_g, ln1_b, l2_W, l2_b, ln2_g, ln2_b, out_W, out_b):
    raise NotImplementedError("write your pallas kernel here")


# speedup vs baseline: 1.3653x; 1.3653x over previous
import functools

import jax
import jax.numpy as jnp
from jax import lax
from jax.experimental import pallas as pl
from jax.experimental.pallas import tpu as pltpu
from jax.experimental.pallas import tpu_sc as plsc

N_NODES = 50000
N_EDGES = 200000
N_GRAPHS = 2048
N_PAIRS = 1024
IN_DIM = 66
HID = 128
NUM_LABELS = 792

NP = 50176
CHUNK = 12544
TRASH = 128
ACC_ROWS = CHUNK + TRASH
E_PAD = 200704
EDGE_BLK = 128
N_ECHUNK = E_PAD // (16 * EDGE_BLK)
ZROWS = 264

G_ACC = N_GRAPHS + TRASH
NODE_BLK = 112
N_NCHUNK = NP // (16 * NODE_BLK)

LBL_PAD = 896


def _sc_mesh():
    return plsc.VectorSubcoreMesh(core_axis_name="c", subcore_axis_name="s")


def _edge_agg_body(src_hbm, dst_hbm, x_hbm, agg_hbm,
                   srcbuf, dstbuf, locbuf, rows, acc, sem):
    c = lax.axis_index("c")
    s = lax.axis_index("s")

    @pl.loop(0, 2)
    def _(p):
        base = (2 * c + p) * CHUNK

        zv = jnp.zeros((16,), jnp.float32)
        @pl.loop(0, EDGE_BLK)
        def _(i):
            for v in range(8):
                rows[i, pl.ds(16 * v, 16)] = zv
        for z in range(6):
            pltpu.sync_copy(rows, acc.at[pl.ds(s * 792 + z * EDGE_BLK,
                                               EDGE_BLK)])
        pltpu.sync_copy(rows.at[pl.ds(0, 24)],
                        acc.at[pl.ds(s * 792 + 768, 24)])
        plsc.subcore_barrier()

        @pl.loop(0, N_ECHUNK)
        def _(j):
            eoff = (s * N_ECHUNK + j) * EDGE_BLK
            pltpu.sync_copy(src_hbm.at[pl.ds(eoff, EDGE_BLK)], srcbuf)
            pltpu.sync_copy(dst_hbm.at[pl.ds(eoff, EDGE_BLK)], dstbuf)
            pltpu.async_copy(x_hbm.at[srcbuf], rows, sem).wait()
            lane = lax.iota(jnp.int32, 16)
            for v in range(8):
                d = dstbuf[pl.ds(16 * v, 16)]
                loc = d - base
                ok = (loc >= 0) & (loc < CHUNK)
                tloc = CHUNK + lane + 16 * v
                locbuf[0, pl.ds(16 * v, 16)] = jnp.where(ok, loc, tloc)
            pltpu.sync_copy(rows, acc.at[locbuf.at[0]], add=True)

        plsc.subcore_barrier()
        pltpu.sync_copy(acc.at[pl.ds(s * 784, 784)],
                        agg_hbm.at[pl.ds(base + s * 784, 784)])
        plsc.subcore_barrier()


def _edge_agg(src, dst, x):
    kern = pl.kernel(
        _edge_agg_body,
        out_type=jax.ShapeDtypeStruct((NP, HID), jnp.float32),
        mesh=_sc_mesh(),
        scratch_types=[
            pltpu.VMEM((EDGE_BLK,), jnp.int32),
            pltpu.VMEM((EDGE_BLK,), jnp.int32),
            pltpu.VMEM((1, EDGE_BLK), jnp.int32),
            pltpu.VMEM((EDGE_BLK, HID), jnp.float32),
            pltpu.VMEM_SHARED((ACC_ROWS, HID), jnp.float32),
            pltpu.SemaphoreType.DMA,
        ],
    )
    return kern(src, dst, x)


def _readout_body(x_hbm, n2g_hbm, h_hbm, t_hbm, gf_hbm, zh_hbm, zt_hbm,
                  gbuf, rowbuf, ibuf, prow, acc, sem):
    c = lax.axis_index("c")
    s = lax.axis_index("s")

    @pl.when(c == 0)
    def _():
        zv = jnp.zeros((16,), jnp.float32)
        @pl.loop(0, NODE_BLK)
        def _(i):
            for v in range(8):
                rowbuf[i, pl.ds(16 * v, 16)] = zv
        pltpu.sync_copy(rowbuf, acc.at[pl.ds(s * 136, NODE_BLK)])
        pltpu.sync_copy(rowbuf.at[pl.ds(0, 24)],
                        acc.at[pl.ds(s * 136 + NODE_BLK, 24)])
        plsc.subcore_barrier()

        @pl.loop(0, N_NCHUNK)
        def _(j):
            noff = s * (N_NCHUNK * NODE_BLK) + j * NODE_BLK
            pltpu.sync_copy(n2g_hbm.at[pl.ds(noff, NODE_BLK)], gbuf.at[0])
            pltpu.sync_copy(x_hbm.at[pl.ds(noff, NODE_BLK)], rowbuf)
            pltpu.sync_copy(rowbuf, acc.at[gbuf.at[0]], add=True)

        plsc.subcore_barrier()
        pltpu.sync_copy(acc.at[pl.ds(s * 128, 128)],
                        gf_hbm.at[pl.ds(s * 128, 128)])
        plsc.subcore_barrier()

        pltpu.sync_copy(h_hbm.at[pl.ds(s * 64, 64)], ibuf)
        pltpu.async_copy(gf_hbm.at[ibuf], prow, sem).wait()
        pltpu.sync_copy(prow, zh_hbm.at[pl.ds(s * 64, 64)])
        pltpu.sync_copy(t_hbm.at[pl.ds(s * 64, 64)], ibuf)
        pltpu.async_copy(gf_hbm.at[ibuf], prow, sem).wait()
        pltpu.sync_copy(prow, zt_hbm.at[pl.ds(s * 64, 64)])


def _readout(x, n2g, h_inds, t_inds):
    kern = pl.kernel(
        _readout_body,
        out_type=(
            jax.ShapeDtypeStruct((N_GRAPHS, HID), jnp.float32),
            jax.ShapeDtypeStruct((N_PAIRS, HID), jnp.float32),
            jax.ShapeDtypeStruct((N_PAIRS, HID), jnp.float32),
        ),
        mesh=_sc_mesh(),
        scratch_types=[
            pltpu.VMEM((1, NODE_BLK), jnp.int32),
            pltpu.VMEM((NODE_BLK, HID), jnp.float32),
            pltpu.VMEM((64,), jnp.int32),
            pltpu.VMEM((64, HID), jnp.float32),
            pltpu.VMEM_SHARED((G_ACC, HID), jnp.float32),
            pltpu.SemaphoreType.DMA,
        ],
    )
    return kern(x, n2g, h_inds, t_inds)


def _conv_mlp_body(x_ref, agg_ref, wa_ref, ba_ref, wb_ref, bb_ref, o_ref):
    h = x_ref[...] + agg_ref[...]
    h = jnp.dot(h, wa_ref[...], preferred_element_type=jnp.float32)
    h = jnp.maximum(h + ba_ref[...], 0.0)
    h = jnp.dot(h, wb_ref[...], preferred_element_type=jnp.float32)
    o_ref[...] = jnp.maximum(h + bb_ref[...], 0.0)


def _conv_mlp(x, agg, wa, ba, wb, bb):
    blk = 512
    grid = NP // blk
    return pl.pallas_call(
        _conv_mlp_body,
        grid=(grid,),
        in_specs=[
            pl.BlockSpec((blk, HID), lambda i: (i, 0)),
            pl.BlockSpec((blk, HID), lambda i: (i, 0)),
            pl.BlockSpec((HID, HID), lambda i: (0, 0)),
            pl.BlockSpec((1, HID), lambda i: (0, 0)),
            pl.BlockSpec((HID, HID), lambda i: (0, 0)),
            pl.BlockSpec((1, HID), lambda i: (0, 0)),
        ],
        out_specs=pl.BlockSpec((blk, HID), lambda i: (i, 0)),
        out_shape=jax.ShapeDtypeStruct((NP, HID), jnp.float32),
    )(x, agg, wa, ba, wb, bb)


def _ln(x, g, b):
    mu = jnp.mean(x, axis=-1, keepdims=True)
    var = jnp.mean((x - mu) * (x - mu), axis=-1, keepdims=True)
    return (x - mu) * jax.lax.rsqrt(var + 1e-5) * g + b


def _head_body(zh_ref, zt_ref, wh_ref, wt_ref, b1_ref, g1_ref, be1_ref,
               w2_ref, b2_ref, g2_ref, be2_ref, wo_ref, bo_ref, lbl_ref,
               o_ref):
    h = (jnp.dot(zh_ref[...], wh_ref[...], preferred_element_type=jnp.float32)
         + jnp.dot(zt_ref[...], wt_ref[...], preferred_element_type=jnp.float32)
         + b1_ref[...])
    h = jnp.maximum(_ln(h, g1_ref[...], be1_ref[...]), 0.0)
    h = jnp.dot(h, w2_ref[...], preferred_element_type=jnp.float32) + b2_ref[...]
    h = jnp.maximum(_ln(h, g2_ref[...], be2_ref[...]), 0.0)
    out = jnp.dot(h, wo_ref[...], preferred_element_type=jnp.float32) + bo_ref[...]
    cols = lax.broadcasted_iota(jnp.int32, (N_PAIRS, LBL_PAD), 1)
    sel = jnp.where(cols == lbl_ref[...], out, 0.0)
    o_ref[...] = jnp.sum(sel, axis=-1, keepdims=True)


def _head(zh, zt, wh, wt, b1, g1, be1, w2, b2, g2, be2, wo, bo, lbl):
    return pl.pallas_call(
        _head_body,
        out_shape=jax.ShapeDtypeStruct((N_PAIRS, 1), jnp.float32),
    )(zh, zt, wh, wt, b1, g1, be1, w2, b2, g2, be2, wo, bo, lbl)


def kernel(node_feature, edge_index, node2graph, h_inds, t_inds, labels,
           conv0_Wa, conv0_ba, conv0_Wb, conv0_bb,
           conv1_Wa, conv1_ba, conv1_Wb, conv1_bb,
           conv2_Wa, conv2_ba, conv2_Wb, conv2_bb,
           fc1_W, fc1_b, ln1_g, ln1_b,
           l2_W, l2_b, ln2_g, ln2_b,
           out_W, out_b):
    f32 = jnp.float32

    x = jnp.zeros((NP, HID), f32).at[:N_NODES, :IN_DIM].set(node_feature)
    src = jnp.full((E_PAD,), 0, jnp.int32).at[:N_EDGES].set(edge_index[0])
    sent = 1 << 20
    dst = jnp.full((E_PAD,), sent, jnp.int32).at[:N_EDGES].set(edge_index[1])
    n2g = jnp.concatenate(
        [node2graph,
         N_GRAPHS + (jnp.arange(NP - N_NODES, dtype=jnp.int32) % TRASH)])

    wa0 = jnp.zeros((HID, HID), f32).at[:IN_DIM].set(conv0_Wa)
    convs = [(wa0, conv0_ba, conv0_Wb, conv0_bb),
             (conv1_Wa, conv1_ba, conv1_Wb, conv1_bb),
             (conv2_Wa, conv2_ba, conv2_Wb, conv2_bb)]

    for wa, ba, wb, bb in convs:
        agg = _edge_agg(src, dst, x)
        x = _conv_mlp(x, agg, wa, ba.reshape(1, HID), wb, bb.reshape(1, HID))

    _gf, zh, zt = _readout(x, n2g, h_inds, t_inds)

    wo = jnp.zeros((1024, LBL_PAD), f32).at[:, :NUM_LABELS].set(out_W)
    bo = jnp.zeros((1, LBL_PAD), f32).at[0, :NUM_LABELS].set(out_b)
    res = _head(zh, zt,
                fc1_W[:HID], fc1_W[HID:], fc1_b.reshape(1, -1),
                ln1_g.reshape(1, -1), ln1_b.reshape(1, -1),
                l2_W, l2_b.reshape(1, -1),
                ln2_g.reshape(1, -1), ln2_b.reshape(1, -1),
                wo, bo, labels.reshape(N_PAIRS, 1))
    return res[:, 0]


# speedup vs baseline: 1.8179x; 1.3315x over previous
import functools

import jax
import jax.numpy as jnp
from jax import lax
from jax.experimental import pallas as pl
from jax.experimental.pallas import tpu as pltpu
from jax.experimental.pallas import tpu_sc as plsc

N_NODES = 50000
N_EDGES = 200000
N_GRAPHS = 2048
N_PAIRS = 1024
IN_DIM = 66
HID = 128
NUM_LABELS = 792

NP = 50176
CHUNK = 12544
TRASH = 128
ACC_ROWS = CHUNK + TRASH
E_PAD = 200704
EDGE_BLK = 128
N_ECHUNK = E_PAD // (16 * EDGE_BLK)
ZROWS = 264

G_ACC = N_GRAPHS + TRASH
NODE_BLK = 112
N_NCHUNK = NP // (16 * NODE_BLK)

LBL_PAD = 896


def _vgather(x, idx):
    return lax.gather(
        x, idx[:, None],
        lax.GatherDimensionNumbers(offset_dims=(), collapsed_slice_dims=(0,),
                                   start_index_map=(0,)),
        (1,), mode=lax.GatherScatterMode.PROMISE_IN_BOUNDS)


def _sc_mesh():
    return plsc.VectorSubcoreMesh(core_axis_name="c", subcore_axis_name="s")


EBLK = 112
NCH = E_PAD // (16 * EBLK)
ETRASH = 64


def _edge_agg_body(src_hbm, dst_hbm, x_hbm, agg_hbm,
                   srcA, srcB, locA, locB, rowsA, rowsB, acc, gA, gB):
    c = lax.axis_index("c")
    s = lax.axis_index("s")
    lane = lax.iota(jnp.int32, 16)

    def prep(slot_src, slot_loc, rowsX, gX, cn, base):
        eoff = s * (NCH * EBLK) + cn * EBLK
        pltpu.sync_copy(src_hbm.at[pl.ds(eoff, EBLK)], slot_src)
        pltpu.sync_copy(dst_hbm.at[pl.ds(eoff, EBLK)], slot_loc.at[0])
        for v in range(EBLK // 16):
            d = slot_loc[0, pl.ds(16 * v, 16)]
            loc = d - base
            ok = (loc >= 0) & (loc < CHUNK)
            tloc = CHUNK + ((lane + 16 * v) & (ETRASH - 1))
            slot_loc[0, pl.ds(16 * v, 16)] = jnp.where(ok, loc, tloc)
        pltpu.async_copy(x_hbm.at[slot_src], rowsX, gX)

    def fire(slot_src, slot_loc, rowsX, gX):
        pltpu.make_async_copy(x_hbm.at[slot_src], rowsX, gX).wait()
        pltpu.sync_copy(rowsX, acc.at[slot_loc.at[0]], add=True)

    @pl.loop(0, 2)
    def _(p):
        base = (2 * c + p) * CHUNK

        zv = jnp.zeros((16,), jnp.float32)
        @pl.loop(0, EBLK)
        def _(i):
            for v in range(8):
                rowsA[i, pl.ds(16 * v, 16)] = zv
        for z in range(7):
            pltpu.sync_copy(rowsA, acc.at[pl.ds(s * 792 + z * EBLK, EBLK)])
        pltpu.sync_copy(rowsA.at[pl.ds(0, 8)],
                        acc.at[pl.ds(s * 792 + 784, 8)])
        plsc.subcore_barrier()

        prep(srcA, locA, rowsA, gA, 0, base)
        prep(srcB, locB, rowsB, gB, 1, base)

        @pl.loop(0, NCH // 2)
        def _(jj):
            fire(srcA, locA, rowsA, gA)
            @pl.when(jj < NCH // 2 - 1)
            def _():
                prep(srcA, locA, rowsA, gA, 2 * jj + 2, base)
            fire(srcB, locB, rowsB, gB)
            @pl.when(jj < NCH // 2 - 1)
            def _():
                prep(srcB, locB, rowsB, gB, 2 * jj + 3, base)

        plsc.subcore_barrier()
        pltpu.sync_copy(acc.at[pl.ds(s * 784, 784)],
                        agg_hbm.at[pl.ds(base + s * 784, 784)])
        plsc.subcore_barrier()


def _edge_agg(src, dst, x):
    kern = pl.kernel(
        _edge_agg_body,
        out_type=jax.ShapeDtypeStruct((NP, HID), jnp.float32),
        mesh=_sc_mesh(),
        scratch_types=[
            pltpu.VMEM((EBLK,), jnp.int32),
            pltpu.VMEM((EBLK,), jnp.int32),
            pltpu.VMEM((1, EBLK), jnp.int32),
            pltpu.VMEM((1, EBLK), jnp.int32),
            pltpu.VMEM((EBLK, HID), jnp.float32),
            pltpu.VMEM((EBLK, HID), jnp.float32),
            pltpu.VMEM_SHARED((ACC_ROWS, HID), jnp.float32),
            pltpu.SemaphoreType.DMA,
            pltpu.SemaphoreType.DMA,
        ],
    )
    return kern(src, dst, x)


def _readout_body(x_hbm, n2g_hbm, h_hbm, t_hbm, gf_hbm, zh_hbm, zt_hbm,
                  gbuf, rowbuf, ibuf, prow, acc, sem):
    c = lax.axis_index("c")
    s = lax.axis_index("s")

    @pl.when(c == 0)
    def _():
        zv = jnp.zeros((16,), jnp.float32)
        @pl.loop(0, NODE_BLK)
        def _(i):
            for v in range(8):
                rowbuf[i, pl.ds(16 * v, 16)] = zv
        pltpu.sync_copy(rowbuf, acc.at[pl.ds(s * 136, NODE_BLK)])
        pltpu.sync_copy(rowbuf.at[pl.ds(0, 24)],
                        acc.at[pl.ds(s * 136 + NODE_BLK, 24)])
        plsc.subcore_barrier()

        @pl.loop(0, N_NCHUNK)
        def _(j):
            noff = s * (N_NCHUNK * NODE_BLK) + j * NODE_BLK
            pltpu.sync_copy(n2g_hbm.at[pl.ds(noff, NODE_BLK)], gbuf.at[0])
            pltpu.sync_copy(x_hbm.at[pl.ds(noff, NODE_BLK)], rowbuf)
            pltpu.sync_copy(rowbuf, acc.at[gbuf.at[0]], add=True)

        plsc.subcore_barrier()
        pltpu.sync_copy(acc.at[pl.ds(s * 128, 128)],
                        gf_hbm.at[pl.ds(s * 128, 128)])
        plsc.subcore_barrier()

        pltpu.sync_copy(h_hbm.at[pl.ds(s * 64, 64)], ibuf)
        pltpu.async_copy(gf_hbm.at[ibuf], prow, sem).wait()
        pltpu.sync_copy(prow, zh_hbm.at[pl.ds(s * 64, 64)])
        pltpu.sync_copy(t_hbm.at[pl.ds(s * 64, 64)], ibuf)
        pltpu.async_copy(gf_hbm.at[ibuf], prow, sem).wait()
        pltpu.sync_copy(prow, zt_hbm.at[pl.ds(s * 64, 64)])


def _readout(x, n2g, h_inds, t_inds):
    kern = pl.kernel(
        _readout_body,
        out_type=(
            jax.ShapeDtypeStruct((N_GRAPHS, HID), jnp.float32),
            jax.ShapeDtypeStruct((N_PAIRS, HID), jnp.float32),
            jax.ShapeDtypeStruct((N_PAIRS, HID), jnp.float32),
        ),
        mesh=_sc_mesh(),
        scratch_types=[
            pltpu.VMEM((1, NODE_BLK), jnp.int32),
            pltpu.VMEM((NODE_BLK, HID), jnp.float32),
            pltpu.VMEM((64,), jnp.int32),
            pltpu.VMEM((64, HID), jnp.float32),
            pltpu.VMEM_SHARED((G_ACC, HID), jnp.float32),
            pltpu.SemaphoreType.DMA,
        ],
    )
    return kern(x, n2g, h_inds, t_inds)


def _conv_mlp_body(x_ref, agg_ref, wa_ref, ba_ref, wb_ref, bb_ref, o_ref):
    h = x_ref[...] + agg_ref[...]
    h = jnp.dot(h, wa_ref[...], preferred_element_type=jnp.float32)
    h = jnp.maximum(h + ba_ref[...], 0.0)
    h = jnp.dot(h, wb_ref[...], preferred_element_type=jnp.float32)
    o_ref[...] = jnp.maximum(h + bb_ref[...], 0.0)


def _conv_mlp(x, agg, wa, ba, wb, bb):
    blk = 512
    grid = NP // blk
    return pl.pallas_call(
        _conv_mlp_body,
        grid=(grid,),
        in_specs=[
            pl.BlockSpec((blk, HID), lambda i: (i, 0)),
            pl.BlockSpec((blk, HID), lambda i: (i, 0)),
            pl.BlockSpec((HID, HID), lambda i: (0, 0)),
            pl.BlockSpec((1, HID), lambda i: (0, 0)),
            pl.BlockSpec((HID, HID), lambda i: (0, 0)),
            pl.BlockSpec((1, HID), lambda i: (0, 0)),
        ],
        out_specs=pl.BlockSpec((blk, HID), lambda i: (i, 0)),
        out_shape=jax.ShapeDtypeStruct((NP, HID), jnp.float32),
    )(x, agg, wa, ba, wb, bb)


def _ln(x, g, b):
    mu = jnp.mean(x, axis=-1, keepdims=True)
    var = jnp.mean((x - mu) * (x - mu), axis=-1, keepdims=True)
    return (x - mu) * jax.lax.rsqrt(var + 1e-5) * g + b


def _head_body(zh_ref, zt_ref, wh_ref, wt_ref, b1_ref, g1_ref, be1_ref,
               w2_ref, b2_ref, g2_ref, be2_ref, wo_ref, bo_ref, lbl_ref,
               o_ref):
    h = (jnp.dot(zh_ref[...], wh_ref[...], preferred_element_type=jnp.float32)
         + jnp.dot(zt_ref[...], wt_ref[...], preferred_element_type=jnp.float32)
         + b1_ref[...])
    h = jnp.maximum(_ln(h, g1_ref[...], be1_ref[...]), 0.0)
    h = jnp.dot(h, w2_ref[...], preferred_element_type=jnp.float32) + b2_ref[...]
    h = jnp.maximum(_ln(h, g2_ref[...], be2_ref[...]), 0.0)
    out = jnp.dot(h, wo_ref[...], preferred_element_type=jnp.float32) + bo_ref[...]
    cols = lax.broadcasted_iota(jnp.int32, (N_PAIRS, LBL_PAD), 1)
    sel = jnp.where(cols == lbl_ref[...], out, 0.0)
    o_ref[...] = jnp.sum(sel, axis=-1, keepdims=True)


def _head(zh, zt, wh, wt, b1, g1, be1, w2, b2, g2, be2, wo, bo, lbl):
    return pl.pallas_call(
        _head_body,
        out_shape=jax.ShapeDtypeStruct((N_PAIRS, 1), jnp.float32),
    )(zh, zt, wh, wt, b1, g1, be1, w2, b2, g2, be2, wo, bo, lbl)


def kernel(node_feature, edge_index, node2graph, h_inds, t_inds, labels,
           conv0_Wa, conv0_ba, conv0_Wb, conv0_bb,
           conv1_Wa, conv1_ba, conv1_Wb, conv1_bb,
           conv2_Wa, conv2_ba, conv2_Wb, conv2_bb,
           fc1_W, fc1_b, ln1_g, ln1_b,
           l2_W, l2_b, ln2_g, ln2_b,
           out_W, out_b):
    f32 = jnp.float32

    x = jnp.zeros((NP, HID), f32).at[:N_NODES, :IN_DIM].set(node_feature)
    src = jnp.full((E_PAD,), 0, jnp.int32).at[:N_EDGES].set(edge_index[0])
    sent = 1 << 20
    dst = jnp.full((E_PAD,), sent, jnp.int32).at[:N_EDGES].set(edge_index[1])
    n2g = jnp.concatenate(
        [node2graph,
         N_GRAPHS + (jnp.arange(NP - N_NODES, dtype=jnp.int32) % TRASH)])

    wa0 = jnp.zeros((HID, HID), f32).at[:IN_DIM].set(conv0_Wa)
    convs = [(wa0, conv0_ba, conv0_Wb, conv0_bb),
             (conv1_Wa, conv1_ba, conv1_Wb, conv1_bb),
             (conv2_Wa, conv2_ba, conv2_Wb, conv2_bb)]

    for wa, ba, wb, bb in convs:
        agg = _edge_agg(src, dst, x)
        x = _conv_mlp(x, agg, wa, ba.reshape(1, HID), wb, bb.reshape(1, HID))

    _gf, zh, zt = _readout(x, n2g, h_inds, t_inds)

    wo = jnp.zeros((1024, LBL_PAD), f32).at[:, :NUM_LABELS].set(out_W)
    bo = jnp.zeros((1, LBL_PAD), f32).at[0, :NUM_LABELS].set(out_b)
    res = _head(zh, zt,
                fc1_W[:HID], fc1_W[HID:], fc1_b.reshape(1, -1),
                ln1_g.reshape(1, -1), ln1_b.reshape(1, -1),
                l2_W, l2_b.reshape(1, -1),
                ln2_g.reshape(1, -1), ln2_b.reshape(1, -1),
                wo, bo, labels.reshape(N_PAIRS, 1))
    return res[:, 0]


# speedup vs baseline: 2.0378x; 1.1209x over previous
import functools

import jax
import jax.numpy as jnp
from jax import lax
from jax.experimental import pallas as pl
from jax.experimental.pallas import tpu as pltpu
from jax.experimental.pallas import tpu_sc as plsc

N_NODES = 50000
N_EDGES = 200000
N_GRAPHS = 2048
N_PAIRS = 1024
IN_DIM = 66
HID = 128
NUM_LABELS = 792

NP = 50176
CHUNK = 12544
TRASH = 128
ACC_ROWS = CHUNK + TRASH
E_PAD = 200704
EDGE_BLK = 128
N_ECHUNK = E_PAD // (16 * EDGE_BLK)
ZROWS = 264

G_ACC = N_GRAPHS + TRASH
NODE_BLK = 112
N_NCHUNK = NP // (16 * NODE_BLK)

LBL_PAD = 896


def _vgather(x, idx):
    return lax.gather(
        x, idx[:, None],
        lax.GatherDimensionNumbers(offset_dims=(), collapsed_slice_dims=(0,),
                                   start_index_map=(0,)),
        (1,), mode=lax.GatherScatterMode.PROMISE_IN_BOUNDS)


def _sc_mesh():
    return plsc.VectorSubcoreMesh(core_axis_name="c", subcore_axis_name="s")


EBLK = 112
NCH = E_PAD // (16 * EBLK)
ETRASH = 64


def _edge_agg_body(ei_hbm, x_hbm, agg_hbm,
                   ebufA, ebufB, rowsA, rowsB, acc, gA, gB):
    c = lax.axis_index("c")
    s = lax.axis_index("s")
    lane = lax.iota(jnp.int32, 16)

    def prep(ebufX, rowsX, gX, cn, base):
        cid = s * NCH + cn
        pltpu.sync_copy(ei_hbm.at[cid], ebufX)
        for v in range(EBLK // 16):
            d = ebufX[1, pl.ds(16 * v, 16)]
            loc = d - base
            ok = (loc >= 0) & (loc < CHUNK)
            tloc = CHUNK + s * 8 + ((lane + 16 * v) & 7)
            ebufX[1, pl.ds(16 * v, 16)] = jnp.where(ok, loc, tloc)
        pltpu.async_copy(x_hbm.at[ebufX.at[0]], rowsX, gX)

    def fire(ebufX, rowsX, gX):
        pltpu.make_async_copy(x_hbm.at[ebufX.at[0]], rowsX, gX).wait()
        pltpu.sync_copy(rowsX, acc.at[ebufX.at[1]], add=True)

    @pl.loop(0, 2)
    def _(p):
        base = (2 * c + p) * CHUNK

        zv = jnp.zeros((16,), jnp.float32)
        @pl.loop(0, EBLK)
        def _(i):
            for v in range(8):
                rowsA[i, pl.ds(16 * v, 16)] = zv
        for z in range(7):
            pltpu.sync_copy(rowsA, acc.at[pl.ds(s * 792 + z * EBLK, EBLK)])
        pltpu.sync_copy(rowsA.at[pl.ds(0, 8)],
                        acc.at[pl.ds(s * 792 + 784, 8)])
        plsc.subcore_barrier()

        prep(ebufA, rowsA, gA, 0, base)
        prep(ebufB, rowsB, gB, 1, base)

        @pl.loop(0, NCH // 2)
        def _(jj):
            fire(ebufA, rowsA, gA)
            @pl.when(jj < NCH // 2 - 1)
            def _():
                prep(ebufA, rowsA, gA, 2 * jj + 2, base)
            fire(ebufB, rowsB, gB)
            @pl.when(jj < NCH // 2 - 1)
            def _():
                prep(ebufB, rowsB, gB, 2 * jj + 3, base)

        plsc.subcore_barrier()
        pltpu.sync_copy(acc.at[pl.ds(s * 784, 784)],
                        agg_hbm.at[pl.ds(base + s * 784, 784)])
        plsc.subcore_barrier()


def _edge_agg(ei, x):
    kern = pl.kernel(
        _edge_agg_body,
        out_type=jax.ShapeDtypeStruct((NP, HID), jnp.float32),
        mesh=_sc_mesh(),
        scratch_types=[
            pltpu.VMEM((2, EBLK), jnp.int32),
            pltpu.VMEM((2, EBLK), jnp.int32),
            pltpu.VMEM((EBLK, HID), jnp.float32),
            pltpu.VMEM((EBLK, HID), jnp.float32),
            pltpu.VMEM_SHARED((ACC_ROWS, HID), jnp.float32),
            pltpu.SemaphoreType.DMA,
            pltpu.SemaphoreType.DMA,
        ],
    )
    return kern(ei, x)


def _readout_body(x_hbm, n2g_hbm, h_hbm, t_hbm, gf_hbm, zh_hbm, zt_hbm,
                  gbuf, rowbuf, ibuf, prow, acc, sem):
    c = lax.axis_index("c")
    s = lax.axis_index("s")

    @pl.when(c == 0)
    def _():
        zv = jnp.zeros((16,), jnp.float32)
        @pl.loop(0, NODE_BLK)
        def _(i):
            for v in range(8):
                rowbuf[i, pl.ds(16 * v, 16)] = zv
        pltpu.sync_copy(rowbuf, acc.at[pl.ds(s * 136, NODE_BLK)])
        pltpu.sync_copy(rowbuf.at[pl.ds(0, 24)],
                        acc.at[pl.ds(s * 136 + NODE_BLK, 24)])
        plsc.subcore_barrier()

        @pl.loop(0, N_NCHUNK)
        def _(j):
            noff = s * (N_NCHUNK * NODE_BLK) + j * NODE_BLK
            pltpu.sync_copy(n2g_hbm.at[pl.ds(noff, NODE_BLK)], gbuf.at[0])
            pltpu.sync_copy(x_hbm.at[pl.ds(noff, NODE_BLK)], rowbuf)
            pltpu.sync_copy(rowbuf, acc.at[gbuf.at[0]], add=True)

        plsc.subcore_barrier()
        pltpu.sync_copy(acc.at[pl.ds(s * 128, 128)],
                        gf_hbm.at[pl.ds(s * 128, 128)])
        plsc.subcore_barrier()

        pltpu.sync_copy(h_hbm.at[pl.ds(s * 64, 64)], ibuf)
        pltpu.async_copy(gf_hbm.at[ibuf], prow, sem).wait()
        pltpu.sync_copy(prow, zh_hbm.at[pl.ds(s * 64, 64)])
        pltpu.sync_copy(t_hbm.at[pl.ds(s * 64, 64)], ibuf)
        pltpu.async_copy(gf_hbm.at[ibuf], prow, sem).wait()
        pltpu.sync_copy(prow, zt_hbm.at[pl.ds(s * 64, 64)])


def _readout(x, n2g, h_inds, t_inds):
    kern = pl.kernel(
        _readout_body,
        out_type=(
            jax.ShapeDtypeStruct((N_GRAPHS, HID), jnp.float32),
            jax.ShapeDtypeStruct((N_PAIRS, HID), jnp.float32),
            jax.ShapeDtypeStruct((N_PAIRS, HID), jnp.float32),
        ),
        mesh=_sc_mesh(),
        scratch_types=[
            pltpu.VMEM((1, NODE_BLK), jnp.int32),
            pltpu.VMEM((NODE_BLK, HID), jnp.float32),
            pltpu.VMEM((64,), jnp.int32),
            pltpu.VMEM((64, HID), jnp.float32),
            pltpu.VMEM_SHARED((G_ACC, HID), jnp.float32),
            pltpu.SemaphoreType.DMA,
        ],
    )
    return kern(x, n2g, h_inds, t_inds)


def _conv_mlp_body(x_ref, agg_ref, wa_ref, ba_ref, wb_ref, bb_ref, o_ref):
    h = x_ref[...] + agg_ref[...]
    h = jnp.dot(h, wa_ref[...], preferred_element_type=jnp.float32)
    h = jnp.maximum(h + ba_ref[...], 0.0)
    h = jnp.dot(h, wb_ref[...], preferred_element_type=jnp.float32)
    o_ref[...] = jnp.maximum(h + bb_ref[...], 0.0)


def _conv_mlp(x, agg, wa, ba, wb, bb):
    blk = 512
    grid = NP // blk
    return pl.pallas_call(
        _conv_mlp_body,
        grid=(grid,),
        in_specs=[
            pl.BlockSpec((blk, HID), lambda i: (i, 0)),
            pl.BlockSpec((blk, HID), lambda i: (i, 0)),
            pl.BlockSpec((HID, HID), lambda i: (0, 0)),
            pl.BlockSpec((1, HID), lambda i: (0, 0)),
            pl.BlockSpec((HID, HID), lambda i: (0, 0)),
            pl.BlockSpec((1, HID), lambda i: (0, 0)),
        ],
        out_specs=pl.BlockSpec((blk, HID), lambda i: (i, 0)),
        out_shape=jax.ShapeDtypeStruct((NP, HID), jnp.float32),
    )(x, agg, wa, ba, wb, bb)


def _ln(x, g, b):
    mu = jnp.mean(x, axis=-1, keepdims=True)
    var = jnp.mean((x - mu) * (x - mu), axis=-1, keepdims=True)
    return (x - mu) * jax.lax.rsqrt(var + 1e-5) * g + b


def _head_body(zh_ref, zt_ref, wh_ref, wt_ref, b1_ref, g1_ref, be1_ref,
               w2_ref, b2_ref, g2_ref, be2_ref, wo_ref, bo_ref, lbl_ref,
               o_ref):
    h = (jnp.dot(zh_ref[...], wh_ref[...], preferred_element_type=jnp.float32)
         + jnp.dot(zt_ref[...], wt_ref[...], preferred_element_type=jnp.float32)
         + b1_ref[...])
    h = jnp.maximum(_ln(h, g1_ref[...], be1_ref[...]), 0.0)
    h = jnp.dot(h, w2_ref[...], preferred_element_type=jnp.float32) + b2_ref[...]
    h = jnp.maximum(_ln(h, g2_ref[...], be2_ref[...]), 0.0)
    out = jnp.dot(h, wo_ref[...], preferred_element_type=jnp.float32) + bo_ref[...]
    cols = lax.broadcasted_iota(jnp.int32, (N_PAIRS, LBL_PAD), 1)
    sel = jnp.where(cols == lbl_ref[...], out, 0.0)
    o_ref[...] = jnp.sum(sel, axis=-1, keepdims=True)


def _head(zh, zt, wh, wt, b1, g1, be1, w2, b2, g2, be2, wo, bo, lbl):
    return pl.pallas_call(
        _head_body,
        out_shape=jax.ShapeDtypeStruct((N_PAIRS, 1), jnp.float32),
    )(zh, zt, wh, wt, b1, g1, be1, w2, b2, g2, be2, wo, bo, lbl)


def kernel(node_feature, edge_index, node2graph, h_inds, t_inds, labels,
           conv0_Wa, conv0_ba, conv0_Wb, conv0_bb,
           conv1_Wa, conv1_ba, conv1_Wb, conv1_bb,
           conv2_Wa, conv2_ba, conv2_Wb, conv2_bb,
           fc1_W, fc1_b, ln1_g, ln1_b,
           l2_W, l2_b, ln2_g, ln2_b,
           out_W, out_b):
    f32 = jnp.float32

    x = jnp.zeros((NP, HID), f32).at[:N_NODES, :IN_DIM].set(node_feature)
    sent = 1 << 20
    src = jnp.full((E_PAD,), 0, jnp.int32).at[:N_EDGES].set(edge_index[0])
    dst = jnp.full((E_PAD,), sent, jnp.int32).at[:N_EDGES].set(edge_index[1])
    ei = jnp.stack([src.reshape(-1, EBLK), dst.reshape(-1, EBLK)], axis=1)
    n2g = jnp.concatenate(
        [node2graph,
         N_GRAPHS + (jnp.arange(NP - N_NODES, dtype=jnp.int32) % TRASH)])

    wa0 = jnp.zeros((HID, HID), f32).at[:IN_DIM].set(conv0_Wa)
    convs = [(wa0, conv0_ba, conv0_Wb, conv0_bb),
             (conv1_Wa, conv1_ba, conv1_Wb, conv1_bb),
             (conv2_Wa, conv2_ba, conv2_Wb, conv2_bb)]

    for wa, ba, wb, bb in convs:
        agg = _edge_agg(ei, x)
        x = _conv_mlp(x, agg, wa, ba.reshape(1, HID), wb, bb.reshape(1, HID))

    _gf, zh, zt = _readout(x, n2g, h_inds, t_inds)

    wo = jnp.zeros((1024, LBL_PAD), f32).at[:, :NUM_LABELS].set(out_W)
    bo = jnp.zeros((1, LBL_PAD), f32).at[0, :NUM_LABELS].set(out_b)
    res = _head(zh, zt,
                fc1_W[:HID], fc1_W[HID:], fc1_b.reshape(1, -1),
                ln1_g.reshape(1, -1), ln1_b.reshape(1, -1),
                l2_W, l2_b.reshape(1, -1),
                ln2_g.reshape(1, -1), ln2_b.reshape(1, -1),
                wo, bo, labels.reshape(N_PAIRS, 1))
    return res[:, 0]
